# Initial kernel scaffold; baseline (speedup 1.0000x reference)
import jax
import jax.numpy as jnp
from jax.experimental import pallas as pl


def kernel(inputs, alpha, beta):
    raise NotImplementedError("write your pallas kernel here")


# speedup vs baseline: 7.0109x; 7.0109x over previous
import functools

import jax
import jax.numpy as jnp
from jax import lax
from jax.experimental import pallas as pl
from jax.experimental.pallas import tpu as pltpu
from jax.experimental.pallas import tpu_sc as plsc

_NFEATS = 64
_NMEM = 128
_BATCH = 4096
_LANES = 16
_NVREG = _NMEM // _LANES
_NWORKERS = 32
_ROWS_PER_W = _BATCH // _NWORKERS
_ROW = _NFEATS * _NMEM


def _sort16(x, asc):
  if asc:
    return jnp.sort(x)
  s, _ = plsc.sort_key_val(x, x, descending=True)
  return s


def _ce(x, y, asc):
  lo = jnp.minimum(x, y)
  hi = jnp.maximum(x, y)
  return (lo, hi) if asc else (hi, lo)


def _sort128(v):
  v = [_sort16(v[r], r % 2 == 0) for r in range(_NVREG)]
  for g in range(4):
    asc = g % 2 == 0
    a, b = _ce(v[2 * g], v[2 * g + 1], asc)
    v[2 * g], v[2 * g + 1] = _sort16(a, asc), _sort16(b, asc)
  for h in range(2):
    asc = h == 0
    base = 4 * h
    for j in (0, 1):
      v[base + j], v[base + 2 + j] = _ce(v[base + j], v[base + 2 + j], asc)
    for j in (0, 2):
      v[base + j], v[base + j + 1] = _ce(v[base + j], v[base + j + 1], asc)
    for j in range(4):
      v[base + j] = _sort16(v[base + j], asc)
  for j in range(4):
    v[j], v[j + 4] = _ce(v[j], v[j + 4], True)
  for j in (0, 1, 4, 5):
    v[j], v[j + 2] = _ce(v[j], v[j + 2], True)
  for j in (0, 2, 4, 6):
    v[j], v[j + 1] = _ce(v[j], v[j + 1], True)
  return [_sort16(x, True) for x in v]


def _body(in_hbm, alpha_hbm, beta_hbm, out_hbm, row_v, out_v, alpha_v, beta_v):
  cid = lax.axis_index("c")
  sid = lax.axis_index("s")
  wid = sid * 2 + cid
  base_row = wid * _ROWS_PER_W

  pltpu.sync_copy(alpha_hbm, alpha_v)
  pltpu.sync_copy(beta_hbm, beta_v)

  iota64 = lax.iota(jnp.int32, _LANES) * _NFEATS

  @pl.loop(0, _ROWS_PER_W)
  def _row(rr):
    row = base_row + rr
    pltpu.sync_copy(in_hbm.at[row], row_v)

    @pl.loop(0, _NFEATS)
    def _feat(i):
      a = alpha_v[pl.ds(i * _LANES, _LANES)]
      b = beta_v[pl.ds(i * _LANES, _LANES)]
      v = []
      for r in range(_NVREG):
        idx = iota64 + (i + r * _LANES * _NFEATS)
        x = plsc.load_gather(row_v, [idx])
        v.append(x * a + b)
      v = _sort128(v)
      for r in range(_NVREG):
        out_v[pl.ds(i * _NMEM + r * _LANES, _LANES)] = v[r]

    pltpu.sync_copy(out_v, out_hbm.at[row])


def kernel(inputs, alpha, beta):
  alpha_rep = jnp.repeat(alpha, _LANES)
  beta_rep = jnp.repeat(beta, _LANES)
  mesh = plsc.VectorSubcoreMesh(core_axis_name="c", subcore_axis_name="s")
  f = pl.kernel(
      _body,
      out_type=jax.ShapeDtypeStruct((_BATCH, _ROW), jnp.float32),
      mesh=mesh,
      compiler_params=pltpu.CompilerParams(needs_layout_passes=False),
      scratch_types=[
          pltpu.VMEM((_ROW,), jnp.float32),
          pltpu.VMEM((_ROW,), jnp.float32),
          pltpu.VMEM((_NFEATS * _LANES,), jnp.float32),
          pltpu.VMEM((_NFEATS * _LANES,), jnp.float32),
      ],
  )
  return f(inputs, alpha_rep, beta_rep)
